# baseline (speedup 1.0000x reference)
import functools

import jax
import jax.numpy as jnp
from jax import lax
from jax.experimental import pallas as pl
from jax.experimental.pallas import tpu as pltpu
from jax.experimental.pallas import tpu_sc as plsc

B, L, D = 16, 4096, 128
HALF = L // 2

_mesh = plsc.VectorSubcoreMesh(core_axis_name="c", subcore_axis_name="s")


@functools.partial(
    pl.kernel,
    out_type=jax.ShapeDtypeStruct((B, L, D), jnp.float32),
    mesh=_mesh,
)
def _sc_copy(x_hbm, o_hbm):
    wid = lax.axis_index("s") * 2 + lax.axis_index("c")
    row = wid // 2
    base = (wid % 2) * HALF
    pltpu.sync_copy(
        x_hbm.at[row, pl.ds(base, HALF)],
        o_hbm.at[row, pl.ds(base, HALF)],
    )


def kernel(sequences, seq_lens):
    return _sc_copy(sequences), seq_lens


# speedup vs baseline: 20.3231x; 20.3231x over previous
import functools

import jax
import jax.numpy as jnp
from jax import lax
from jax.experimental import pallas as pl
from jax.experimental.pallas import tpu as pltpu
from jax.experimental.pallas import tpu_sc as plsc

AUG_T = 1024
B, L, D = 16, 4096, 128
HALF = L // 2
CPOS = 256
NCH = HALF // CPOS
MAXZ = -(-CPOS // 10) + 1

_mesh = plsc.VectorSubcoreMesh(core_axis_name="c", subcore_axis_name="s")


@functools.partial(
    pl.kernel,
    out_type=jax.ShapeDtypeStruct((B, L, D), jnp.float32),
    mesh=_mesh,
    scratch_types=[
        pltpu.VMEM((CPOS, D), jnp.float32),
        pltpu.VMEM((32,), jnp.int32),
    ],
)
def _sc_aug(x_hbm, lens_hbm, o_hbm, buf, lens_v):
    wid = lax.axis_index("s") * 2 + lax.axis_index("c")
    row = wid // 2
    base0 = (wid % 2) * HALF

    pltpu.sync_copy(lens_hbm, lens_v.at[pl.ds(0, 16)])
    slen = lens_v[pl.ds(row, 16)][0]
    lim_half = jnp.where(slen > AUG_T, jnp.minimum(slen - base0, HALF), 0)

    zeros16 = jnp.zeros((16,), jnp.float32)

    for k in range(NCH):
        cbase = k * CPOS
        pltpu.sync_copy(x_hbm.at[row, pl.ds(base0 + cbase, CPOS)], buf)

        first = (10 - (base0 + cbase) % 10) % 10
        limit = jnp.clip(lim_half - cbase, 0, CPOS)

        def zbody(j, _, first=first, limit=limit):
            p = first + 10 * j

            @pl.when(p < limit)
            def _z():
                for i in range(D // 16):
                    buf[p, pl.ds(16 * i, 16)] = zeros16

            return 0

        lax.fori_loop(0, MAXZ, zbody, 0)
        pltpu.sync_copy(buf, o_hbm.at[row, pl.ds(base0 + cbase, CPOS)])


def kernel(sequences, seq_lens):
    return _sc_aug(sequences, seq_lens), seq_lens
